# baseline (speedup 1.0000x reference)
import functools

import jax
import jax.numpy as jnp
from jax import lax
from jax.experimental import pallas as pl
from jax.experimental.pallas import tpu as pltpu
from jax.experimental.pallas import tpu_sc as plsc

IN_FEATS = 16
OUT_FEATS = 128
N_NODES = 10000
N_EDGES = 640000

NC = 2
NS = 16
NW = NC * NS

CHUNK = 128
N_CHUNKS = N_EDGES // CHUNK
CHUNKS_PER_W = -(-N_CHUNKS // NW)
N_PAD = 10112
ROWS_PER_SUB = N_PAD // NS
NBUF = 4
ROUNDS = -(-CHUNKS_PER_W // NBUF)


def _sc_scatter(dst1d, ef4):
    mesh = plsc.VectorSubcoreMesh(core_axis_name="c", subcore_axis_name="s")

    @functools.partial(
        pl.kernel,
        out_type=(
            jax.ShapeDtypeStruct((NC, N_PAD, IN_FEATS), jnp.float32),
            jax.ShapeDtypeStruct((NC, N_PAD, IN_FEATS), jnp.float32),
        ),
        mesh=mesh,
        compiler_params=pltpu.CompilerParams(
            use_tc_tiling_on_sc=False, needs_layout_passes=False),
        scratch_types=[
            [pltpu.VMEM((CHUNK,), jnp.int32) for _ in range(NBUF)],
            [pltpu.VMEM((IN_FEATS, CHUNK), jnp.float32) for _ in range(NBUF)],
            [pltpu.VMEM((CHUNK, IN_FEATS), jnp.float32) for _ in range(NBUF)],
            pltpu.VMEM((CHUNK, IN_FEATS), jnp.float32),
            pltpu.VMEM((ROWS_PER_SUB, IN_FEATS), jnp.float32),
            pltpu.VMEM_SHARED((N_PAD, IN_FEATS), jnp.float32),
            pltpu.VMEM_SHARED((N_PAD, IN_FEATS), jnp.float32),
            [pltpu.SemaphoreType.DMA for _ in range(NBUF)],
            [pltpu.SemaphoreType.DMA for _ in range(NBUF)],
        ],
    )
    def k(dst_hbm, feat_hbm, out_acc, out_deg,
          idx_v, fmaj_v, emaj_v, ones_v, zeros_v, acc_s, deg_s, lsem, ssem):
        c = lax.axis_index("c")
        s = lax.axis_index("s")
        w = s * NC + c

        def fill(i, _):
            zeros_v[i, :] = jnp.zeros((IN_FEATS,), jnp.float32)
            return 0
        lax.fori_loop(0, ROWS_PER_SUB, fill, 0)

        def fill1(i, _):
            ones_v[i, :] = jnp.ones((IN_FEATS,), jnp.float32)
            return 0
        lax.fori_loop(0, CHUNK, fill1, 0)

        pltpu.sync_copy(zeros_v, acc_s.at[pl.ds(s * ROWS_PER_SUB, ROWS_PER_SUB), :])
        pltpu.sync_copy(zeros_v, deg_s.at[pl.ds(s * ROWS_PER_SUB, ROWS_PER_SUB), :])
        plsc.subcore_barrier()

        lane = lax.iota(jnp.int32, 16)

        def start_load(i, b):
            j = w + i * NW

            @pl.when(j < N_CHUNKS)
            def _():
                pltpu.async_copy(dst_hbm.at[pl.ds(j * CHUNK, CHUNK)], idx_v[b],
                                 lsem[b])
                pltpu.async_copy(feat_hbm.at[0, j],
                                 fmaj_v[b].at[pl.ds(0, 8), :], lsem[b])
                pltpu.async_copy(feat_hbm.at[1, j],
                                 fmaj_v[b].at[pl.ds(8, 8), :], lsem[b])

        def wait_load(i, b):
            j = w + i * NW

            @pl.when(j < N_CHUNKS)
            def _():
                pltpu.make_async_copy(dst_hbm.at[pl.ds(j * CHUNK, CHUNK)],
                                      idx_v[b], lsem[b]).wait()
                pltpu.make_async_copy(feat_hbm.at[0, j],
                                      fmaj_v[b].at[pl.ds(0, 8), :],
                                      lsem[b]).wait()
                pltpu.make_async_copy(feat_hbm.at[1, j],
                                      fmaj_v[b].at[pl.ds(8, 8), :],
                                      lsem[b]).wait()

        def transpose(i, b):
            j = w + i * NW

            @pl.when(j < N_CHUNKS)
            def _():
                for e in range(CHUNK):
                    v = plsc.load_gather(
                        fmaj_v[b], [lane, jnp.full((16,), e, jnp.int32)])
                    emaj_v[b][e, :] = v

        def start_scatter(i, b):
            j = w + i * NW

            @pl.when(j < N_CHUNKS)
            def _():
                pltpu.async_copy(emaj_v[b], acc_s.at[idx_v[b]], ssem[b], add=True)
                pltpu.async_copy(ones_v, deg_s.at[idx_v[b]], ssem[b], add=True)

        def wait_scatter(i, b):
            j = w + i * NW

            @pl.when(jnp.logical_and(j >= 0, j < N_CHUNKS))
            def _():
                pltpu.make_async_copy(emaj_v[b], acc_s.at[idx_v[b]], ssem[b]).wait()
                pltpu.make_async_copy(ones_v, deg_s.at[idx_v[b]], ssem[b]).wait()

        start_load(0, 0)
        start_load(1, 1)

        def body(r, _):
            for b in range(NBUF):
                i = r * NBUF + b
                wait_load(i, b)
                transpose(i, b)
                start_scatter(i, b)
                br = (b + 2) % NBUF
                wait_scatter(i - 2, br)
                start_load(i + 2, br)
            return 0
        lax.fori_loop(0, ROUNDS, body, 0)

        last = ROUNDS * NBUF
        wait_scatter(last - 2, (last - 2) % NBUF)
        wait_scatter(last - 1, (last - 1) % NBUF)

        plsc.subcore_barrier()
        sl = pl.ds(s * ROWS_PER_SUB, ROWS_PER_SUB)
        pltpu.sync_copy(acc_s.at[sl, :], out_acc.at[c, sl, :])
        pltpu.sync_copy(deg_s.at[sl, :], out_deg.at[c, sl, :])

    return k(dst1d, ef4)


def _tc_finish(acc2p, deg2p, Wbig, B2):
    def body(a_ref, d_ref, w_ref, b2_ref, o_ref):
        h = a_ref[0] + a_ref[1]
        dsum = d_ref[0] + d_ref[1]
        sp = lax.rsqrt(jnp.maximum(dsum, 1.0))
        hs = h * sp
        o_ref[...] = (
            jnp.dot(hs, w_ref[...], preferred_element_type=jnp.float32)
            + jnp.dot(sp, b2_ref[...], preferred_element_type=jnp.float32)
        )

    return pl.pallas_call(
        body,
        out_shape=jax.ShapeDtypeStruct((N_PAD // 8, 8 * OUT_FEATS), jnp.float32),
    )(acc2p, deg2p, Wbig, B2)


def kernel(node_feats, edge_feats, edge_index, W, b):
    del node_feats
    dst1d = edge_index[1].astype(jnp.int32)
    ef4 = jnp.transpose(
        jnp.reshape(jnp.swapaxes(edge_feats, 0, 1), (2, 8, N_CHUNKS, CHUNK)),
        (0, 2, 1, 3))
    acc2, deg2 = _sc_scatter(dst1d, ef4)
    acc2p = acc2.reshape(NC, N_PAD // 8, 8 * IN_FEATS)
    deg2p = deg2.reshape(NC, N_PAD // 8, 8 * IN_FEATS)
    Wbig = jnp.reshape(
        jnp.eye(8, dtype=jnp.float32)[:, None, :, None] * W.T[None, :, None, :],
        (128, 8 * OUT_FEATS))
    sel = jnp.eye(128, dtype=jnp.float32)[:, ::16]
    B2 = jnp.reshape(sel[:, :, None] * b[None, None, :], (128, 8 * OUT_FEATS))
    out1024 = _tc_finish(acc2p, deg2p, Wbig, B2)
    return out1024.reshape(N_PAD, OUT_FEATS)[:N_NODES]


# speedup vs baseline: 1.3996x; 1.3996x over previous
import functools

import jax
import jax.numpy as jnp
from jax import lax
from jax.experimental import pallas as pl
from jax.experimental.pallas import tpu as pltpu
from jax.experimental.pallas import tpu_sc as plsc

IN_FEATS = 16
OUT_FEATS = 128
N_NODES = 10000
N_EDGES = 640000

NC = 2
NS = 16
NW = NC * NS

CHUNK = 128
N_CHUNKS = N_EDGES // CHUNK
CHUNKS_PER_W = -(-N_CHUNKS // NW)
N_PAD = 10112
ROWS_PER_SUB = N_PAD // NS
NBUF = 4
ROUNDS = -(-CHUNKS_PER_W // NBUF)


def _sc_scatter(dst1d, ef4):
    mesh = plsc.VectorSubcoreMesh(core_axis_name="c", subcore_axis_name="s")

    @functools.partial(
        pl.kernel,
        out_type=(
            jax.ShapeDtypeStruct((NC, N_PAD, IN_FEATS), jnp.float32),
            jax.ShapeDtypeStruct((NC, N_PAD, IN_FEATS), jnp.float32),
        ),
        mesh=mesh,
        compiler_params=pltpu.CompilerParams(
            use_tc_tiling_on_sc=False, needs_layout_passes=False),
        scratch_types=[
            [pltpu.VMEM((CHUNK,), jnp.int32) for _ in range(NBUF)],
            [pltpu.VMEM((IN_FEATS, CHUNK + 9), jnp.float32) for _ in range(NBUF)],
            [pltpu.VMEM((CHUNK, IN_FEATS), jnp.float32) for _ in range(NBUF)],
            pltpu.VMEM((CHUNK, IN_FEATS), jnp.float32),
            pltpu.VMEM((ROWS_PER_SUB, IN_FEATS), jnp.float32),
            pltpu.VMEM_SHARED((N_PAD, IN_FEATS), jnp.float32),
            pltpu.VMEM_SHARED((N_PAD, IN_FEATS), jnp.float32),
            [pltpu.SemaphoreType.DMA for _ in range(NBUF)],
            [pltpu.SemaphoreType.DMA for _ in range(NBUF)],
        ],
    )
    def k(dst_hbm, feat_hbm, out_acc, out_deg,
          idx_v, fmaj_v, emaj_v, ones_v, zeros_v, acc_s, deg_s, lsem, ssem):
        c = lax.axis_index("c")
        s = lax.axis_index("s")
        w = s * NC + c

        def fill(i, _):
            zeros_v[i, :] = jnp.zeros((IN_FEATS,), jnp.float32)
            return 0
        lax.fori_loop(0, ROWS_PER_SUB, fill, 0)

        def fill1(i, _):
            ones_v[i, :] = jnp.ones((IN_FEATS,), jnp.float32)
            return 0
        lax.fori_loop(0, CHUNK, fill1, 0)

        pltpu.sync_copy(zeros_v, acc_s.at[pl.ds(s * ROWS_PER_SUB, ROWS_PER_SUB), :])
        pltpu.sync_copy(zeros_v, deg_s.at[pl.ds(s * ROWS_PER_SUB, ROWS_PER_SUB), :])
        plsc.subcore_barrier()

        lane = lax.iota(jnp.int32, 16)

        def start_load(i, b):
            j = w + i * NW

            @pl.when(j < N_CHUNKS)
            def _():
                pltpu.async_copy(dst_hbm.at[pl.ds(j * CHUNK, CHUNK)], idx_v[b],
                                 lsem[b])
                pltpu.async_copy(feat_hbm.at[0, j],
                                 fmaj_v[b].at[pl.ds(0, 8), pl.ds(0, CHUNK)],
                                 lsem[b])
                pltpu.async_copy(feat_hbm.at[1, j],
                                 fmaj_v[b].at[pl.ds(8, 8), pl.ds(0, CHUNK)],
                                 lsem[b])

        def wait_load(i, b):
            j = w + i * NW

            @pl.when(j < N_CHUNKS)
            def _():
                pltpu.make_async_copy(dst_hbm.at[pl.ds(j * CHUNK, CHUNK)],
                                      idx_v[b], lsem[b]).wait()
                pltpu.make_async_copy(feat_hbm.at[0, j],
                                      fmaj_v[b].at[pl.ds(0, 8), pl.ds(0, CHUNK)],
                                      lsem[b]).wait()
                pltpu.make_async_copy(feat_hbm.at[1, j],
                                      fmaj_v[b].at[pl.ds(8, 8), pl.ds(0, CHUNK)],
                                      lsem[b]).wait()

        def transpose(i, b):
            j = w + i * NW

            @pl.when(j < N_CHUNKS)
            def _():
                for e in range(CHUNK):
                    v = plsc.load_gather(
                        fmaj_v[b], [lane, jnp.full((16,), e, jnp.int32)])
                    emaj_v[b][e, :] = v

        def start_scatter(i, b):
            j = w + i * NW

            @pl.when(j < N_CHUNKS)
            def _():
                pltpu.async_copy(emaj_v[b], acc_s.at[idx_v[b]], ssem[b], add=True)
                pltpu.async_copy(ones_v, deg_s.at[idx_v[b]], ssem[b], add=True)

        def wait_scatter(i, b):
            j = w + i * NW

            @pl.when(jnp.logical_and(j >= 0, j < N_CHUNKS))
            def _():
                pltpu.make_async_copy(emaj_v[b], acc_s.at[idx_v[b]], ssem[b]).wait()
                pltpu.make_async_copy(ones_v, deg_s.at[idx_v[b]], ssem[b]).wait()

        start_load(0, 0)
        start_load(1, 1)

        def body(r, _):
            for b in range(NBUF):
                i = r * NBUF + b
                wait_load(i, b)
                transpose(i, b)
                start_scatter(i, b)
                br = (b + 2) % NBUF
                wait_scatter(i - 2, br)
                start_load(i + 2, br)
            return 0
        lax.fori_loop(0, ROUNDS, body, 0)

        last = ROUNDS * NBUF
        wait_scatter(last - 2, (last - 2) % NBUF)
        wait_scatter(last - 1, (last - 1) % NBUF)

        plsc.subcore_barrier()
        sl = pl.ds(s * ROWS_PER_SUB, ROWS_PER_SUB)
        pltpu.sync_copy(acc_s.at[sl, :], out_acc.at[c, sl, :])
        pltpu.sync_copy(deg_s.at[sl, :], out_deg.at[c, sl, :])

    return k(dst1d, ef4)


def _tc_finish(acc2p, deg2p, Wbig, B2):
    def body(a_ref, d_ref, w_ref, b2_ref, o_ref):
        h = a_ref[0] + a_ref[1]
        dsum = d_ref[0] + d_ref[1]
        sp = lax.rsqrt(jnp.maximum(dsum, 1.0))
        hs = h * sp
        o_ref[...] = (
            jnp.dot(hs, w_ref[...], preferred_element_type=jnp.float32)
            + jnp.dot(sp, b2_ref[...], preferred_element_type=jnp.float32)
        )

    return pl.pallas_call(
        body,
        out_shape=jax.ShapeDtypeStruct((N_PAD // 8, 8 * OUT_FEATS), jnp.float32),
    )(acc2p, deg2p, Wbig, B2)


def kernel(node_feats, edge_feats, edge_index, W, b):
    del node_feats
    dst1d = edge_index[1].astype(jnp.int32)
    ef4 = jnp.transpose(
        jnp.reshape(jnp.swapaxes(edge_feats, 0, 1), (2, 8, N_CHUNKS, CHUNK)),
        (0, 2, 1, 3))
    acc2, deg2 = _sc_scatter(dst1d, ef4)
    acc2p = acc2.reshape(NC, N_PAD // 8, 8 * IN_FEATS)
    deg2p = deg2.reshape(NC, N_PAD // 8, 8 * IN_FEATS)
    Wbig = jnp.reshape(
        jnp.eye(8, dtype=jnp.float32)[:, None, :, None] * W.T[None, :, None, :],
        (128, 8 * OUT_FEATS))
    sel = jnp.eye(128, dtype=jnp.float32)[:, ::16]
    B2 = jnp.reshape(sel[:, :, None] * b[None, None, :], (128, 8 * OUT_FEATS))
    out1024 = _tc_finish(acc2p, deg2p, Wbig, B2)
    return out1024.reshape(N_PAD, OUT_FEATS)[:N_NODES]


# speedup vs baseline: 1.4002x; 1.0004x over previous
import functools

import jax
import jax.numpy as jnp
from jax import lax
from jax.experimental import pallas as pl
from jax.experimental.pallas import tpu as pltpu
from jax.experimental.pallas import tpu_sc as plsc

IN_FEATS = 16
OUT_FEATS = 128
N_NODES = 10000
N_EDGES = 640000

NC = 2
NS = 16
NW = NC * NS

CHUNK = 128
N_CHUNKS = N_EDGES // CHUNK
CHUNKS_PER_W = -(-N_CHUNKS // NW)
N_PAD = 10112
ROWS_PER_SUB = N_PAD // NS
NBUF = 4
ROUNDS = -(-CHUNKS_PER_W // NBUF)


def _sc_scatter(dst1d, ef4):
    mesh = plsc.VectorSubcoreMesh(core_axis_name="c", subcore_axis_name="s")

    @functools.partial(
        pl.kernel,
        out_type=(
            jax.ShapeDtypeStruct((NC, N_PAD, IN_FEATS), jnp.float32),
            jax.ShapeDtypeStruct((NC, N_PAD, IN_FEATS), jnp.float32),
        ),
        mesh=mesh,
        compiler_params=pltpu.CompilerParams(
            use_tc_tiling_on_sc=False, needs_layout_passes=False),
        scratch_types=[
            [pltpu.VMEM((CHUNK,), jnp.int32) for _ in range(NBUF)],
            [pltpu.VMEM((IN_FEATS, CHUNK + 9), jnp.float32) for _ in range(NBUF)],
            [pltpu.VMEM((CHUNK, IN_FEATS), jnp.float32) for _ in range(NBUF)],
            pltpu.VMEM((CHUNK, IN_FEATS), jnp.float32),
            pltpu.VMEM((ROWS_PER_SUB, IN_FEATS), jnp.float32),
            pltpu.VMEM_SHARED((N_PAD, IN_FEATS), jnp.float32),
            pltpu.VMEM_SHARED((N_PAD, IN_FEATS), jnp.float32),
            [pltpu.SemaphoreType.DMA for _ in range(NBUF)],
            [pltpu.SemaphoreType.DMA for _ in range(NBUF)],
        ],
    )
    def k(dst_hbm, feat_hbm, out_acc, out_deg,
          idx_v, fmaj_v, emaj_v, ones_v, zeros_v, acc_s, deg_s, lsem, ssem):
        c = lax.axis_index("c")
        s = lax.axis_index("s")
        w = s * NC + c

        def fill(i, _):
            zeros_v[i, :] = jnp.zeros((IN_FEATS,), jnp.float32)
            return 0
        lax.fori_loop(0, ROWS_PER_SUB, fill, 0)

        def fill1(i, _):
            ones_v[i, :] = jnp.ones((IN_FEATS,), jnp.float32)
            return 0
        lax.fori_loop(0, CHUNK, fill1, 0)

        pltpu.sync_copy(zeros_v, acc_s.at[pl.ds(s * ROWS_PER_SUB, ROWS_PER_SUB), :])
        pltpu.sync_copy(zeros_v, deg_s.at[pl.ds(s * ROWS_PER_SUB, ROWS_PER_SUB), :])
        plsc.subcore_barrier()

        lane = lax.iota(jnp.int32, 16)

        def start_load(i, b):
            j = w + i * NW

            @pl.when(j < N_CHUNKS)
            def _():
                pltpu.async_copy(dst_hbm.at[pl.ds(j * CHUNK, CHUNK)], idx_v[b],
                                 lsem[b])
                pltpu.async_copy(feat_hbm.at[0, j],
                                 fmaj_v[b].at[pl.ds(0, 8), pl.ds(0, CHUNK)],
                                 lsem[b])
                pltpu.async_copy(feat_hbm.at[1, j],
                                 fmaj_v[b].at[pl.ds(8, 8), pl.ds(0, CHUNK)],
                                 lsem[b])

        def wait_load(i, b):
            j = w + i * NW

            @pl.when(j < N_CHUNKS)
            def _():
                pltpu.make_async_copy(dst_hbm.at[pl.ds(j * CHUNK, CHUNK)],
                                      idx_v[b], lsem[b]).wait()
                pltpu.make_async_copy(feat_hbm.at[0, j],
                                      fmaj_v[b].at[pl.ds(0, 8), pl.ds(0, CHUNK)],
                                      lsem[b]).wait()
                pltpu.make_async_copy(feat_hbm.at[1, j],
                                      fmaj_v[b].at[pl.ds(8, 8), pl.ds(0, CHUNK)],
                                      lsem[b]).wait()

        def transpose(i, b):
            j = w + i * NW

            @pl.when(j < N_CHUNKS)
            def _():
                col = jnp.zeros((16,), jnp.int32)
                one = jnp.ones((16,), jnp.int32)
                for e in range(CHUNK):
                    v = plsc.load_gather(fmaj_v[b], [lane, col])
                    emaj_v[b][e, :] = v
                    col = col + one

        def start_scatter(i, b):
            j = w + i * NW

            @pl.when(j < N_CHUNKS)
            def _():
                pltpu.async_copy(emaj_v[b], acc_s.at[idx_v[b]], ssem[b], add=True)
                pltpu.async_copy(ones_v, deg_s.at[idx_v[b]], ssem[b], add=True)

        def wait_scatter(i, b):
            j = w + i * NW

            @pl.when(jnp.logical_and(j >= 0, j < N_CHUNKS))
            def _():
                pltpu.make_async_copy(emaj_v[b], acc_s.at[idx_v[b]], ssem[b]).wait()
                pltpu.make_async_copy(ones_v, deg_s.at[idx_v[b]], ssem[b]).wait()

        start_load(0, 0)
        start_load(1, 1)

        def body(r, _):
            for b in range(NBUF):
                i = r * NBUF + b
                wait_load(i, b)
                transpose(i, b)
                start_scatter(i, b)
                br = (b + 2) % NBUF
                wait_scatter(i - 2, br)
                start_load(i + 2, br)
            return 0
        lax.fori_loop(0, ROUNDS, body, 0)

        last = ROUNDS * NBUF
        wait_scatter(last - 2, (last - 2) % NBUF)
        wait_scatter(last - 1, (last - 1) % NBUF)

        plsc.subcore_barrier()
        sl = pl.ds(s * ROWS_PER_SUB, ROWS_PER_SUB)
        pltpu.sync_copy(acc_s.at[sl, :], out_acc.at[c, sl, :])
        pltpu.sync_copy(deg_s.at[sl, :], out_deg.at[c, sl, :])

    return k(dst1d, ef4)


def _tc_finish(acc2p, deg2p, Wbig, B2):
    def body(a_ref, d_ref, w_ref, b2_ref, o_ref):
        h = a_ref[0] + a_ref[1]
        dsum = d_ref[0] + d_ref[1]
        sp = lax.rsqrt(jnp.maximum(dsum, 1.0))
        hs = h * sp
        o_ref[...] = (
            jnp.dot(hs, w_ref[...], preferred_element_type=jnp.float32)
            + jnp.dot(sp, b2_ref[...], preferred_element_type=jnp.float32)
        )

    return pl.pallas_call(
        body,
        out_shape=jax.ShapeDtypeStruct((N_PAD // 8, 8 * OUT_FEATS), jnp.float32),
    )(acc2p, deg2p, Wbig, B2)


def kernel(node_feats, edge_feats, edge_index, W, b):
    del node_feats
    dst1d = edge_index[1].astype(jnp.int32)
    ef4 = jnp.transpose(
        jnp.reshape(jnp.swapaxes(edge_feats, 0, 1), (2, 8, N_CHUNKS, CHUNK)),
        (0, 2, 1, 3))
    acc2, deg2 = _sc_scatter(dst1d, ef4)
    acc2p = acc2.reshape(NC, N_PAD // 8, 8 * IN_FEATS)
    deg2p = deg2.reshape(NC, N_PAD // 8, 8 * IN_FEATS)
    Wbig = jnp.reshape(
        jnp.eye(8, dtype=jnp.float32)[:, None, :, None] * W.T[None, :, None, :],
        (128, 8 * OUT_FEATS))
    sel = jnp.eye(128, dtype=jnp.float32)[:, ::16]
    B2 = jnp.reshape(sel[:, :, None] * b[None, None, :], (128, 8 * OUT_FEATS))
    out1024 = _tc_finish(acc2p, deg2p, Wbig, B2)
    return out1024.reshape(N_PAD, OUT_FEATS)[:N_NODES]
